# Initial kernel scaffold; baseline (speedup 1.0000x reference)
import jax
import jax.numpy as jnp
from jax.experimental import pallas as pl


def kernel(x, edge_index, Wl1, bl1, Wr1, Wl2, bl2, Wr2, Wc, bc):
    raise NotImplementedError("write your pallas kernel here")


# speedup vs baseline: 7.0923x; 7.0923x over previous
import functools

import jax
import jax.numpy as jnp
from jax import lax
from jax.experimental import pallas as pl
from jax.experimental.pallas import tpu as pltpu
from jax.experimental.pallas import tpu_sc as plsc

N = 10000
E = 320000
IN = 128
H = 64
OUT = 64
C = 40

NC = 2
NS = 16
NW = NC * NS
BATCH = 128
EPT = 10112
STEPS = EPT // BATCH
E_PAD = EPT * NW
N_PAD = 10112
STRIPE = N_PAD // NS
ROWS_BLK = 1000


def _tc_in_proj(x, Wl1, Wr1):
    def body(x_ref, wl_ref, wr_ref, o1_ref, o2_ref):
        xb = x_ref[...]
        dn = (((1,), (1,)), ((), ()))
        o1_ref[...] = lax.dot_general(xb, wl_ref[...], dn,
                                      preferred_element_type=jnp.float32)
        o2_ref[...] = lax.dot_general(xb, wr_ref[...], dn,
                                      preferred_element_type=jnp.float32)

    grid = (N // ROWS_BLK,)
    return pl.pallas_call(
        body,
        grid=grid,
        in_specs=[
            pl.BlockSpec((ROWS_BLK, IN), lambda i: (i, 0)),
            pl.BlockSpec((H, IN), lambda i: (0, 0)),
            pl.BlockSpec((H, IN), lambda i: (0, 0)),
        ],
        out_specs=[
            pl.BlockSpec((ROWS_BLK, H), lambda i: (i, 0)),
            pl.BlockSpec((ROWS_BLK, H), lambda i: (i, 0)),
        ],
        out_shape=[
            jax.ShapeDtypeStruct((N, H), jnp.float32),
            jax.ShapeDtypeStruct((N, H), jnp.float32),
        ],
    )(x, Wl1, Wr1)


def _sc_segment_sum(table, src_r, dst_r, z_acc, ones_b, z_cnt):
    D = table.shape[1]
    count = ones_b is not None
    mesh = plsc.VectorSubcoreMesh(core_axis_name="c", subcore_axis_name="s")

    out_type = [jax.ShapeDtypeStruct((NC, N_PAD, D), jnp.float32)]
    scratch = [
        pltpu.VMEM((STEPS, BATCH), jnp.int32),
        pltpu.VMEM((STEPS, BATCH), jnp.int32),
        pltpu.VMEM((BATCH, D), jnp.float32),
        pltpu.VMEM_SHARED((N_PAD, D), jnp.float32),
    ]
    if count:
        out_type.append(jax.ShapeDtypeStruct((NC, N_PAD, 8), jnp.float32))
        scratch.append(pltpu.VMEM((BATCH, 8), jnp.float32))
        scratch.append(pltpu.VMEM_SHARED((N_PAD, 8), jnp.float32))

    @functools.partial(
        pl.kernel,
        mesh=mesh,
        out_type=out_type,
        scratch_types=scratch,
        compiler_params=pltpu.CompilerParams(use_tc_tiling_on_sc=False),
    )
    def k(*refs):
        if count:
            (table_h, src_h, dst_h, zacc_h, ones_h, zcnt_h,
             out_h, cnt_h, src_v, dst_v, rows_v, acc_s, ones_v, cacc_s) = refs
        else:
            (table_h, src_h, dst_h, zacc_h,
             out_h, src_v, dst_v, rows_v, acc_s) = refs
        c = lax.axis_index("c")
        s = lax.axis_index("s")
        wid = s * NC + c
        r0 = s * STRIPE

        pltpu.sync_copy(src_h.at[wid], src_v)
        pltpu.sync_copy(dst_h.at[wid], dst_v)
        pltpu.sync_copy(zacc_h.at[pl.ds(r0, STRIPE)], acc_s.at[pl.ds(r0, STRIPE)])
        if count:
            pltpu.sync_copy(ones_h, ones_v)
            pltpu.sync_copy(zcnt_h.at[pl.ds(r0, STRIPE)],
                            cacc_s.at[pl.ds(r0, STRIPE)])
        plsc.subcore_barrier()

        def step(j, carry):
            pltpu.sync_copy(table_h.at[src_v.at[j]], rows_v)
            pltpu.sync_copy(rows_v, acc_s.at[dst_v.at[j]], add=True)
            if count:
                pltpu.sync_copy(ones_v, cacc_s.at[dst_v.at[j]], add=True)
            return carry

        lax.fori_loop(0, STEPS, step, 0)
        plsc.subcore_barrier()

        pltpu.sync_copy(acc_s.at[pl.ds(r0, STRIPE)],
                        out_h.at[c, pl.ds(r0, STRIPE)])
        if count:
            pltpu.sync_copy(cacc_s.at[pl.ds(r0, STRIPE)],
                            cnt_h.at[c, pl.ds(r0, STRIPE)])

    if count:
        res = k(table, src_r, dst_r, z_acc, ones_b, z_cnt)
    else:
        res = k(table, src_r, dst_r, z_acc)
    if isinstance(res, (list, tuple)):
        return tuple(res)
    return (res,)


def _tc_mid(f1a, f1b, c8a, c8b, xr1, bl1_2d, Wl2, Wr2, Wc):
    def body(fa_ref, fb_ref, ca_ref, cb_ref, xr_ref, b_ref,
             wl2_ref, wr2_ref, wc_ref, oa_ref, ob_ref):
        aggsum = fa_ref[...] + fb_ref[...]
        cnt = ca_ref[:, 0:1] + cb_ref[:, 0:1]
        inv = 1.0 / jnp.maximum(cnt, 1.0)
        h = jnp.maximum(aggsum * inv + b_ref[0:1, :] + xr_ref[...], 0.0)
        MA = jnp.dot(wc_ref[...], wl2_ref[...],
                     preferred_element_type=jnp.float32)
        MB = jnp.dot(wc_ref[...], wr2_ref[...],
                     preferred_element_type=jnp.float32)
        dn = (((1,), (1,)), ((), ()))
        hA = lax.dot_general(h, MA, dn, preferred_element_type=jnp.float32)
        hB = lax.dot_general(h, MB, dn, preferred_element_type=jnp.float32)
        pad = jnp.zeros((hA.shape[0], 8), jnp.float32)
        oa_ref[...] = jnp.concatenate([hA, pad], axis=1)
        ob_ref[...] = jnp.concatenate([hB, pad], axis=1)

    grid = (N // ROWS_BLK,)
    blk = lambda d: pl.BlockSpec((ROWS_BLK, d), lambda i: (i, 0))
    full = lambda a, b: pl.BlockSpec((a, b), lambda i: (0, 0))
    return pl.pallas_call(
        body,
        grid=grid,
        in_specs=[blk(H), blk(H), blk(8), blk(8), blk(H), full(8, H),
                  full(OUT, H), full(OUT, H), full(C, OUT)],
        out_specs=[blk(48), blk(48)],
        out_shape=[
            jax.ShapeDtypeStruct((N, 48), jnp.float32),
            jax.ShapeDtypeStruct((N, 48), jnp.float32),
        ],
    )(f1a, f1b, c8a, c8b, xr1, bl1_2d, Wl2, Wr2, Wc)


def _tc_out(f2a, f2b, c8a, c8b, hB, bl2_2d, bc_2d, Wc):
    def body(fa_ref, fb_ref, ca_ref, cb_ref, hb_ref, bl_ref, bc_ref, wc_ref,
             o_ref):
        aggsum = fa_ref[...] + fb_ref[...]
        cnt = ca_ref[:, 0:1] + cb_ref[:, 0:1]
        inv = 1.0 / jnp.maximum(cnt, 1.0)
        dn = (((1,), (1,)), ((), ()))
        c2 = lax.dot_general(bl_ref[0:1, :], wc_ref[...], dn,
                             preferred_element_type=jnp.float32)
        o_ref[...] = (aggsum[:, :C] * inv + hb_ref[:, :C]
                      + c2 + bc_ref[0:1, :])

    grid = (N // ROWS_BLK,)
    blk = lambda d: pl.BlockSpec((ROWS_BLK, d), lambda i: (i, 0))
    full = lambda a, b: pl.BlockSpec((a, b), lambda i: (0, 0))
    return pl.pallas_call(
        body,
        grid=grid,
        in_specs=[blk(48), blk(48), blk(8), blk(8), blk(48), full(8, OUT),
                  full(8, C), full(C, OUT)],
        out_specs=blk(C),
        out_shape=jax.ShapeDtypeStruct((N, C), jnp.float32),
    )(f2a, f2b, c8a, c8b, hB, bl2_2d, bc_2d, Wc)


def kernel(x, edge_index, Wl1, bl1, Wr1, Wl2, bl2, Wr2, Wc, bc):
    src = edge_index[0]
    dst = edge_index[1]
    pad = E_PAD - E
    src_r = jnp.concatenate([src, jnp.zeros((pad,), jnp.int32)]
                            ).reshape(NW, STEPS, BATCH)
    dst_r = jnp.concatenate([dst, jnp.full((pad,), N, jnp.int32)]
                            ).reshape(NW, STEPS, BATCH)
    z64 = jnp.zeros((N_PAD, H), jnp.float32)
    z48 = jnp.zeros((N_PAD, 48), jnp.float32)
    z8 = jnp.zeros((N_PAD, 8), jnp.float32)
    ones_b = jnp.ones((BATCH, 8), jnp.float32)
    bl1_2d = jnp.broadcast_to(bl1[None, :], (8, H))
    bl2_2d = jnp.broadcast_to(bl2[None, :], (8, OUT))
    bc_2d = jnp.broadcast_to(bc[None, :], (8, C))

    xw1, xr1 = _tc_in_proj(x, Wl1, Wr1)

    f1_part, cnt_part = _sc_segment_sum(xw1, src_r, dst_r, z64, ones_b, z8)
    f1a = f1_part[0, :N]
    f1b = f1_part[1, :N]
    c8a = cnt_part[0, :N]
    c8b = cnt_part[1, :N]

    hA, hB = _tc_mid(f1a, f1b, c8a, c8b, xr1, bl1_2d, Wl2, Wr2, Wc)

    (f2_part,) = _sc_segment_sum(hA, src_r, dst_r, z48, None, None)
    f2a = f2_part[0, :N]
    f2b = f2_part[1, :N]

    return _tc_out(f2a, f2b, c8a, c8b, hB, bl2_2d, bc_2d, Wc)


# speedup vs baseline: 8.8558x; 1.2487x over previous
import functools

import jax
import jax.numpy as jnp
from jax import lax
from jax.experimental import pallas as pl
from jax.experimental.pallas import tpu as pltpu
from jax.experimental.pallas import tpu_sc as plsc

N = 10000
E = 320000
IN = 128
H = 64
OUT = 64
C = 40

NC = 2
NS = 16
NW = NC * NS
BATCH = 128
EPT = 10112
STEPS = EPT // BATCH
E_PAD = EPT * NW
N_PAD = 10112
STRIPE = N_PAD // NS
ROWS_BLK = 1000


def _tc_in_proj(x, Wl1, Wr1):
    def body(x_ref, wl_ref, wr_ref, o1_ref, o2_ref):
        xb = x_ref[...]
        dn = (((1,), (1,)), ((), ()))
        o1_ref[...] = lax.dot_general(xb, wl_ref[...], dn,
                                      preferred_element_type=jnp.float32)
        o2_ref[...] = lax.dot_general(xb, wr_ref[...], dn,
                                      preferred_element_type=jnp.float32)

    grid = (N // ROWS_BLK,)
    return pl.pallas_call(
        body,
        grid=grid,
        in_specs=[
            pl.BlockSpec((ROWS_BLK, IN), lambda i: (i, 0)),
            pl.BlockSpec((H, IN), lambda i: (0, 0)),
            pl.BlockSpec((H, IN), lambda i: (0, 0)),
        ],
        out_specs=[
            pl.BlockSpec((ROWS_BLK, H), lambda i: (i, 0)),
            pl.BlockSpec((ROWS_BLK, H), lambda i: (i, 0)),
        ],
        out_shape=[
            jax.ShapeDtypeStruct((N, H), jnp.float32),
            jax.ShapeDtypeStruct((N, H), jnp.float32),
        ],
    )(x, Wl1, Wr1)


def _sc_segment_sum(table, src_r, dst_r, z_acc, ones_b, z_cnt):
    D = table.shape[1]
    count = ones_b is not None
    mesh = plsc.VectorSubcoreMesh(core_axis_name="c", subcore_axis_name="s")

    out_type = [jax.ShapeDtypeStruct((NC, N_PAD, D), jnp.float32)]
    scratch = [
        pltpu.VMEM((STEPS, BATCH), jnp.int32),
        pltpu.VMEM((STEPS, BATCH), jnp.int32),
        pltpu.VMEM((2, BATCH, D), jnp.float32),
        pltpu.VMEM_SHARED((N_PAD, D), jnp.float32),
        pltpu.SemaphoreType.DMA,
    ]
    if count:
        out_type.append(jax.ShapeDtypeStruct((NC, N_PAD, 8), jnp.float32))
        scratch.append(pltpu.VMEM((BATCH, 8), jnp.float32))
        scratch.append(pltpu.VMEM_SHARED((N_PAD, 8), jnp.float32))

    @functools.partial(
        pl.kernel,
        mesh=mesh,
        out_type=out_type,
        scratch_types=scratch,
        compiler_params=pltpu.CompilerParams(use_tc_tiling_on_sc=False),
    )
    def k(*refs):
        if count:
            (table_h, src_h, dst_h, zacc_h, ones_h, zcnt_h,
             out_h, cnt_h, src_v, dst_v, rows_v, acc_s, gsem,
             ones_v, cacc_s) = refs
        else:
            (table_h, src_h, dst_h, zacc_h,
             out_h, src_v, dst_v, rows_v, acc_s, gsem) = refs
        c = lax.axis_index("c")
        s = lax.axis_index("s")
        wid = s * NC + c
        r0 = s * STRIPE

        pltpu.sync_copy(src_h.at[wid], src_v)
        pltpu.sync_copy(dst_h.at[wid], dst_v)
        pltpu.sync_copy(zacc_h.at[pl.ds(r0, STRIPE)], acc_s.at[pl.ds(r0, STRIPE)])
        if count:
            pltpu.sync_copy(ones_h, ones_v)
            pltpu.sync_copy(zcnt_h.at[pl.ds(r0, STRIPE)],
                            cacc_s.at[pl.ds(r0, STRIPE)])
        plsc.subcore_barrier()

        pltpu.async_copy(table_h.at[src_v.at[0]], rows_v.at[0], gsem)

        def step(j, carry):
            p = lax.rem(j, 2)
            pn = lax.rem(j + 1, 2)

            @pl.when(j < STEPS - 1)
            def _():
                pltpu.async_copy(table_h.at[src_v.at[j + 1]],
                                 rows_v.at[pn], gsem)

            pltpu.make_async_copy(table_h.at[src_v.at[j]],
                                  rows_v.at[p], gsem).wait()
            pltpu.sync_copy(rows_v.at[p], acc_s.at[dst_v.at[j]], add=True)
            if count:
                pltpu.sync_copy(ones_v, cacc_s.at[dst_v.at[j]], add=True)
            return carry

        lax.fori_loop(0, STEPS, step, 0)
        plsc.subcore_barrier()

        pltpu.sync_copy(acc_s.at[pl.ds(r0, STRIPE)],
                        out_h.at[c, pl.ds(r0, STRIPE)])
        if count:
            pltpu.sync_copy(cacc_s.at[pl.ds(r0, STRIPE)],
                            cnt_h.at[c, pl.ds(r0, STRIPE)])

    if count:
        res = k(table, src_r, dst_r, z_acc, ones_b, z_cnt)
    else:
        res = k(table, src_r, dst_r, z_acc)
    if isinstance(res, (list, tuple)):
        return tuple(res)
    return (res,)


def _tc_mid(f1a, f1b, c8a, c8b, xr1, bl1_2d, Wl2, Wr2, Wc):
    def body(fa_ref, fb_ref, ca_ref, cb_ref, xr_ref, b_ref,
             wl2_ref, wr2_ref, wc_ref, oa_ref, ob_ref):
        aggsum = fa_ref[...] + fb_ref[...]
        cnt = ca_ref[:, 0:1] + cb_ref[:, 0:1]
        inv = 1.0 / jnp.maximum(cnt, 1.0)
        h = jnp.maximum(aggsum * inv + b_ref[0:1, :] + xr_ref[...], 0.0)
        MA = jnp.dot(wc_ref[...], wl2_ref[...],
                     preferred_element_type=jnp.float32)
        MB = jnp.dot(wc_ref[...], wr2_ref[...],
                     preferred_element_type=jnp.float32)
        dn = (((1,), (1,)), ((), ()))
        hA = lax.dot_general(h, MA, dn, preferred_element_type=jnp.float32)
        hB = lax.dot_general(h, MB, dn, preferred_element_type=jnp.float32)
        pad = jnp.zeros((hA.shape[0], 8), jnp.float32)
        oa_ref[...] = jnp.concatenate([hA, pad], axis=1)
        ob_ref[...] = jnp.concatenate([hB, pad], axis=1)

    grid = (N // ROWS_BLK,)
    blk = lambda d: pl.BlockSpec((ROWS_BLK, d), lambda i: (i, 0))
    full = lambda a, b: pl.BlockSpec((a, b), lambda i: (0, 0))
    return pl.pallas_call(
        body,
        grid=grid,
        in_specs=[blk(H), blk(H), blk(8), blk(8), blk(H), full(8, H),
                  full(OUT, H), full(OUT, H), full(C, OUT)],
        out_specs=[blk(48), blk(48)],
        out_shape=[
            jax.ShapeDtypeStruct((N, 48), jnp.float32),
            jax.ShapeDtypeStruct((N, 48), jnp.float32),
        ],
    )(f1a, f1b, c8a, c8b, xr1, bl1_2d, Wl2, Wr2, Wc)


def _tc_out(f2a, f2b, c8a, c8b, hB, bl2_2d, bc_2d, Wc):
    def body(fa_ref, fb_ref, ca_ref, cb_ref, hb_ref, bl_ref, bc_ref, wc_ref,
             o_ref):
        aggsum = fa_ref[...] + fb_ref[...]
        cnt = ca_ref[:, 0:1] + cb_ref[:, 0:1]
        inv = 1.0 / jnp.maximum(cnt, 1.0)
        dn = (((1,), (1,)), ((), ()))
        c2 = lax.dot_general(bl_ref[0:1, :], wc_ref[...], dn,
                             preferred_element_type=jnp.float32)
        o_ref[...] = (aggsum[:, :C] * inv + hb_ref[:, :C]
                      + c2 + bc_ref[0:1, :])

    grid = (N // ROWS_BLK,)
    blk = lambda d: pl.BlockSpec((ROWS_BLK, d), lambda i: (i, 0))
    full = lambda a, b: pl.BlockSpec((a, b), lambda i: (0, 0))
    return pl.pallas_call(
        body,
        grid=grid,
        in_specs=[blk(48), blk(48), blk(8), blk(8), blk(48), full(8, OUT),
                  full(8, C), full(C, OUT)],
        out_specs=blk(C),
        out_shape=jax.ShapeDtypeStruct((N, C), jnp.float32),
    )(f2a, f2b, c8a, c8b, hB, bl2_2d, bc_2d, Wc)


def kernel(x, edge_index, Wl1, bl1, Wr1, Wl2, bl2, Wr2, Wc, bc):
    src = edge_index[0]
    dst = edge_index[1]
    pad = E_PAD - E
    src_r = jnp.concatenate([src, jnp.zeros((pad,), jnp.int32)]
                            ).reshape(NW, STEPS, BATCH)
    dst_r = jnp.concatenate([dst, jnp.full((pad,), N, jnp.int32)]
                            ).reshape(NW, STEPS, BATCH)
    z64 = jnp.zeros((N_PAD, H), jnp.float32)
    z48 = jnp.zeros((N_PAD, 48), jnp.float32)
    z8 = jnp.zeros((N_PAD, 8), jnp.float32)
    ones_b = jnp.ones((BATCH, 8), jnp.float32)
    bl1_2d = jnp.broadcast_to(bl1[None, :], (8, H))
    bl2_2d = jnp.broadcast_to(bl2[None, :], (8, OUT))
    bc_2d = jnp.broadcast_to(bc[None, :], (8, C))

    xw1, xr1 = _tc_in_proj(x, Wl1, Wr1)

    f1_part, cnt_part = _sc_segment_sum(xw1, src_r, dst_r, z64, ones_b, z8)
    f1a = f1_part[0, :N]
    f1b = f1_part[1, :N]
    c8a = cnt_part[0, :N]
    c8b = cnt_part[1, :N]

    hA, hB = _tc_mid(f1a, f1b, c8a, c8b, xr1, bl1_2d, Wl2, Wr2, Wc)

    (f2_part,) = _sc_segment_sum(hA, src_r, dst_r, z48, None, None)
    f2a = f2_part[0, :N]
    f2b = f2_part[1, :N]

    return _tc_out(f2a, f2b, c8a, c8b, hB, bl2_2d, bc_2d, Wc)


# speedup vs baseline: 9.3485x; 1.0556x over previous
import functools

import jax
import jax.numpy as jnp
from jax import lax
from jax.experimental import pallas as pl
from jax.experimental.pallas import tpu as pltpu
from jax.experimental.pallas import tpu_sc as plsc

N = 10000
E = 320000
IN = 128
H = 64
OUT = 64
C = 40

NC = 2
NS = 16
NW = NC * NS
BATCH = 128
EPT = 10112
STEPS = EPT // BATCH
E_PAD = EPT * NW
N_PAD = 10112
STRIPE = N_PAD // NS
ROWS_BLK = 1000
NBUF = 4


def _tc_in_proj(x, Wl1, Wr1):
    def body(x_ref, wl_ref, wr_ref, o1_ref, o2_ref):
        xb = x_ref[...]
        dn = (((1,), (1,)), ((), ()))
        o1_ref[...] = lax.dot_general(xb, wl_ref[...], dn,
                                      preferred_element_type=jnp.float32)
        o2_ref[...] = lax.dot_general(xb, wr_ref[...], dn,
                                      preferred_element_type=jnp.float32)

    grid = (N // ROWS_BLK,)
    return pl.pallas_call(
        body,
        grid=grid,
        in_specs=[
            pl.BlockSpec((ROWS_BLK, IN), lambda i: (i, 0)),
            pl.BlockSpec((H, IN), lambda i: (0, 0)),
            pl.BlockSpec((H, IN), lambda i: (0, 0)),
        ],
        out_specs=[
            pl.BlockSpec((ROWS_BLK, H), lambda i: (i, 0)),
            pl.BlockSpec((ROWS_BLK, H), lambda i: (i, 0)),
        ],
        out_shape=[
            jax.ShapeDtypeStruct((N, H), jnp.float32),
            jax.ShapeDtypeStruct((N, H), jnp.float32),
        ],
    )(x, Wl1, Wr1)


def _sc_segment_sum(table, src_r, dst_r, z_acc, ones_b, z_cnt):
    D = table.shape[1]
    count = ones_b is not None
    mesh = plsc.VectorSubcoreMesh(core_axis_name="c", subcore_axis_name="s")

    out_type = [jax.ShapeDtypeStruct((NC, N_PAD, D), jnp.float32)]
    scratch = [
        pltpu.VMEM((STEPS, BATCH), jnp.int32),
        pltpu.VMEM((STEPS, BATCH), jnp.int32),
        pltpu.VMEM((NBUF, BATCH, D), jnp.float32),
        pltpu.VMEM_SHARED((N_PAD, D), jnp.float32),
        pltpu.SemaphoreType.DMA,
    ]
    if count:
        out_type.append(jax.ShapeDtypeStruct((NC, N_PAD, 8), jnp.float32))
        scratch.append(pltpu.VMEM((BATCH, 8), jnp.float32))
        scratch.append(pltpu.VMEM_SHARED((N_PAD, 8), jnp.float32))

    @functools.partial(
        pl.kernel,
        mesh=mesh,
        out_type=out_type,
        scratch_types=scratch,
        compiler_params=pltpu.CompilerParams(use_tc_tiling_on_sc=False),
    )
    def k(*refs):
        if count:
            (table_h, src_h, dst_h, zacc_h, ones_h, zcnt_h,
             out_h, cnt_h, src_v, dst_v, rows_v, acc_s, gsem,
             ones_v, cacc_s) = refs
        else:
            (table_h, src_h, dst_h, zacc_h,
             out_h, src_v, dst_v, rows_v, acc_s, gsem) = refs
        c = lax.axis_index("c")
        s = lax.axis_index("s")
        wid = s * NC + c
        r0 = s * STRIPE

        pltpu.sync_copy(src_h.at[wid], src_v)
        pltpu.sync_copy(dst_h.at[wid], dst_v)
        pltpu.sync_copy(zacc_h.at[pl.ds(r0, STRIPE)], acc_s.at[pl.ds(r0, STRIPE)])
        if count:
            pltpu.sync_copy(ones_h, ones_v)
            pltpu.sync_copy(zcnt_h.at[pl.ds(r0, STRIPE)],
                            cacc_s.at[pl.ds(r0, STRIPE)])
        plsc.subcore_barrier()

        for jj in range(NBUF - 1):
            pltpu.async_copy(table_h.at[src_v.at[jj]], rows_v.at[jj], gsem)

        def step(j, carry):
            p = lax.rem(j, NBUF)

            @pl.when(j < STEPS - (NBUF - 1))
            def _():
                jn = j + NBUF - 1
                pltpu.async_copy(table_h.at[src_v.at[jn]],
                                 rows_v.at[lax.rem(jn, NBUF)], gsem)

            pltpu.make_async_copy(table_h.at[src_v.at[j]],
                                  rows_v.at[p], gsem).wait()
            pltpu.sync_copy(rows_v.at[p], acc_s.at[dst_v.at[j]], add=True)
            if count:
                pltpu.sync_copy(ones_v, cacc_s.at[dst_v.at[j]], add=True)
            return carry

        lax.fori_loop(0, STEPS, step, 0)
        plsc.subcore_barrier()

        pltpu.sync_copy(acc_s.at[pl.ds(r0, STRIPE)],
                        out_h.at[c, pl.ds(r0, STRIPE)])
        if count:
            pltpu.sync_copy(cacc_s.at[pl.ds(r0, STRIPE)],
                            cnt_h.at[c, pl.ds(r0, STRIPE)])

    if count:
        res = k(table, src_r, dst_r, z_acc, ones_b, z_cnt)
    else:
        res = k(table, src_r, dst_r, z_acc)
    if isinstance(res, (list, tuple)):
        return tuple(res)
    return (res,)


def _tc_mid(f1a, f1b, c8a, c8b, xr1, bl1_2d, Wl2, Wr2, Wc):
    def body(fa_ref, fb_ref, ca_ref, cb_ref, xr_ref, b_ref,
             wl2_ref, wr2_ref, wc_ref, oa_ref, ob_ref):
        aggsum = fa_ref[...] + fb_ref[...]
        cnt = ca_ref[:, 0:1] + cb_ref[:, 0:1]
        inv = 1.0 / jnp.maximum(cnt, 1.0)
        h = jnp.maximum(aggsum * inv + b_ref[0:1, :] + xr_ref[...], 0.0)
        MA = jnp.dot(wc_ref[...], wl2_ref[...],
                     preferred_element_type=jnp.float32)
        MB = jnp.dot(wc_ref[...], wr2_ref[...],
                     preferred_element_type=jnp.float32)
        dn = (((1,), (1,)), ((), ()))
        hA = lax.dot_general(h, MA, dn, preferred_element_type=jnp.float32)
        hB = lax.dot_general(h, MB, dn, preferred_element_type=jnp.float32)
        pad = jnp.zeros((hA.shape[0], 8), jnp.float32)
        oa_ref[...] = jnp.concatenate([hA, pad], axis=1)
        ob_ref[...] = jnp.concatenate([hB, pad], axis=1)

    grid = (N // ROWS_BLK,)
    blk = lambda d: pl.BlockSpec((ROWS_BLK, d), lambda i: (i, 0))
    full = lambda a, b: pl.BlockSpec((a, b), lambda i: (0, 0))
    return pl.pallas_call(
        body,
        grid=grid,
        in_specs=[blk(H), blk(H), blk(8), blk(8), blk(H), full(8, H),
                  full(OUT, H), full(OUT, H), full(C, OUT)],
        out_specs=[blk(48), blk(48)],
        out_shape=[
            jax.ShapeDtypeStruct((N, 48), jnp.float32),
            jax.ShapeDtypeStruct((N, 48), jnp.float32),
        ],
    )(f1a, f1b, c8a, c8b, xr1, bl1_2d, Wl2, Wr2, Wc)


def _tc_out(f2a, f2b, c8a, c8b, hB, bl2_2d, bc_2d, Wc):
    def body(fa_ref, fb_ref, ca_ref, cb_ref, hb_ref, bl_ref, bc_ref, wc_ref,
             o_ref):
        aggsum = fa_ref[...] + fb_ref[...]
        cnt = ca_ref[:, 0:1] + cb_ref[:, 0:1]
        inv = 1.0 / jnp.maximum(cnt, 1.0)
        dn = (((1,), (1,)), ((), ()))
        c2 = lax.dot_general(bl_ref[0:1, :], wc_ref[...], dn,
                             preferred_element_type=jnp.float32)
        o_ref[...] = (aggsum[:, :C] * inv + hb_ref[:, :C]
                      + c2 + bc_ref[0:1, :])

    grid = (N // ROWS_BLK,)
    blk = lambda d: pl.BlockSpec((ROWS_BLK, d), lambda i: (i, 0))
    full = lambda a, b: pl.BlockSpec((a, b), lambda i: (0, 0))
    return pl.pallas_call(
        body,
        grid=grid,
        in_specs=[blk(48), blk(48), blk(8), blk(8), blk(48), full(8, OUT),
                  full(8, C), full(C, OUT)],
        out_specs=blk(C),
        out_shape=jax.ShapeDtypeStruct((N, C), jnp.float32),
    )(f2a, f2b, c8a, c8b, hB, bl2_2d, bc_2d, Wc)


def kernel(x, edge_index, Wl1, bl1, Wr1, Wl2, bl2, Wr2, Wc, bc):
    src = edge_index[0]
    dst = edge_index[1]
    pad = E_PAD - E
    src_r = jnp.concatenate([src, jnp.zeros((pad,), jnp.int32)]
                            ).reshape(NW, STEPS, BATCH)
    dst_r = jnp.concatenate([dst, jnp.full((pad,), N, jnp.int32)]
                            ).reshape(NW, STEPS, BATCH)
    z64 = jnp.zeros((N_PAD, H), jnp.float32)
    z48 = jnp.zeros((N_PAD, 48), jnp.float32)
    z8 = jnp.zeros((N_PAD, 8), jnp.float32)
    ones_b = jnp.ones((BATCH, 8), jnp.float32)
    bl1_2d = jnp.broadcast_to(bl1[None, :], (8, H))
    bl2_2d = jnp.broadcast_to(bl2[None, :], (8, OUT))
    bc_2d = jnp.broadcast_to(bc[None, :], (8, C))

    xw1, xr1 = _tc_in_proj(x, Wl1, Wr1)

    f1_part, cnt_part = _sc_segment_sum(xw1, src_r, dst_r, z64, ones_b, z8)
    f1a = f1_part[0, :N]
    f1b = f1_part[1, :N]
    c8a = cnt_part[0, :N]
    c8b = cnt_part[1, :N]

    hA, hB = _tc_mid(f1a, f1b, c8a, c8b, xr1, bl1_2d, Wl2, Wr2, Wc)

    (f2_part,) = _sc_segment_sum(hA, src_r, dst_r, z48, None, None)
    f2a = f2_part[0, :N]
    f2b = f2_part[1, :N]

    return _tc_out(f2a, f2b, c8a, c8b, hB, bl2_2d, bc_2d, Wc)
